# baseline (speedup 1.0000x reference)
import functools

import jax
import jax.numpy as jnp
from jax.experimental.layout import Layout, with_layout_constraint
from jax import lax
from jax.experimental import pallas as pl
from jax.experimental.pallas import tpu as pltpu
from jax.experimental.pallas import tpu_sc as plsc

B = 4096
F = 26
V = 100000
NC = 2
NS = 16
NW = NC * NS
RPW = B // NW
L = 16
EPW = RPW * F


def _lr_body(x_hbm, w_hbm, bias_hbm, out_hbm, xr_v, idx_v, gath_v, acc_v,
             bias_v, sem):
    wid = lax.axis_index("s") * NC + lax.axis_index("c")
    base = wid * RPW

    pltpu.sync_copy(x_hbm.at[pl.ds(base * F, EPW)], xr_v)
    pltpu.sync_copy(bias_hbm, bias_v)

    iota = lax.iota(jnp.int32, L)
    iF = iota * F
    for f in range(F):
        for c in range(RPW // L):
            src = iF + (c * L * F + f)
            idx_v[f, pl.ds(c * L, L)] = plsc.load_gather(xr_v, [src]) + f * V

    copies = [
        pltpu.async_copy(w_hbm.at[idx_v.at[f]], gath_v.at[f], sem)
        for f in range(F)
    ]
    for cp in copies:
        cp.wait()

    bias_r = bias_v[...]
    for c in range(RPW // L):
        sl = pl.ds(c * L, L)
        acc = bias_r
        for f in range(F):
            acc = acc + gath_v[f, sl]
        acc_v[sl] = 1.0 / (1.0 + jnp.exp(-acc))

    pltpu.sync_copy(acc_v, out_hbm.at[pl.ds(base, RPW)])


@jax.jit
def _lr_sc(xflat, wflat, bias16):
    call = functools.partial(
        pl.kernel,
        out_type=jax.ShapeDtypeStruct((B,), jnp.float32),
        mesh=plsc.VectorSubcoreMesh(core_axis_name="c", subcore_axis_name="s"),
        compiler_params=pltpu.CompilerParams(needs_layout_passes=False),
        scratch_types=[
            pltpu.VMEM((EPW,), jnp.int32),
            pltpu.VMEM((F, RPW), jnp.int32),
            pltpu.VMEM((F, RPW), jnp.float32),
            pltpu.VMEM((RPW,), jnp.float32),
            pltpu.VMEM((L,), jnp.float32),
            pltpu.SemaphoreType.DMA,
        ],
    )(_lr_body)
    return call(xflat, wflat, bias16)


def kernel(X, W, bias):
    bias16 = jnp.broadcast_to(bias, (L,))
    wtiled = with_layout_constraint(
        W, Layout(major_to_minor=(2, 0, 1))
    )
    wtiled = lax.optimization_barrier(wtiled)
    out = _lr_sc(X.reshape(B * F), wtiled.reshape(F * V), bias16)
    return out.reshape(B, 1)


# speedup vs baseline: 1.0192x; 1.0192x over previous
import functools

import jax
import jax.numpy as jnp
from jax.experimental.layout import Layout, with_layout_constraint
from jax import lax
from jax.experimental import pallas as pl
from jax.experimental.pallas import tpu as pltpu
from jax.experimental.pallas import tpu_sc as plsc

B = 4096
F = 26
V = 100000
NC = 2
NS = 16
NW = NC * NS
RPW = B // NW
L = 16
EPW = RPW * F


def _lr_body(x_hbm, w_hbm, bias_hbm, out_hbm, xr_v, idx_v, gath_v, acc_v,
             bias_v, sem):
    wid = lax.axis_index("s") * NC + lax.axis_index("c")
    base = wid * RPW

    pltpu.sync_copy(x_hbm.at[pl.ds(base * F, EPW)], xr_v)
    pltpu.sync_copy(bias_hbm, bias_v)

    iota = lax.iota(jnp.int32, L)
    iF = iota * F
    copies = []
    for f in range(F):
        for c in range(RPW // L):
            src = iF + (c * L * F + f)
            idx_v[f, pl.ds(c * L, L)] = plsc.load_gather(xr_v, [src]) + f * V
        copies.append(
            pltpu.async_copy(w_hbm.at[idx_v.at[f]], gath_v.at[f], sem)
        )
    for cp in copies:
        cp.wait()

    bias_r = bias_v[...]
    for c in range(RPW // L):
        sl = pl.ds(c * L, L)
        acc = bias_r
        for f in range(F):
            acc = acc + gath_v[f, sl]
        acc_v[sl] = 1.0 / (1.0 + jnp.exp(-acc))

    pltpu.sync_copy(acc_v, out_hbm.at[pl.ds(base, RPW)])


@jax.jit
def _lr_sc(xflat, wflat, bias16):
    call = functools.partial(
        pl.kernel,
        out_type=jax.ShapeDtypeStruct((B,), jnp.float32),
        mesh=plsc.VectorSubcoreMesh(core_axis_name="c", subcore_axis_name="s"),
        compiler_params=pltpu.CompilerParams(needs_layout_passes=False),
        scratch_types=[
            pltpu.VMEM((EPW,), jnp.int32),
            pltpu.VMEM((F, RPW), jnp.int32),
            pltpu.VMEM((F, RPW), jnp.float32),
            pltpu.VMEM((RPW,), jnp.float32),
            pltpu.VMEM((L,), jnp.float32),
            pltpu.SemaphoreType.DMA,
        ],
    )(_lr_body)
    return call(xflat, wflat, bias16)


def kernel(X, W, bias):
    bias16 = jnp.broadcast_to(bias, (L,))
    wtiled = with_layout_constraint(
        W, Layout(major_to_minor=(2, 0, 1), tiling=((8, 128),))
    )
    wtiled = lax.optimization_barrier(wtiled)
    out = _lr_sc(X.reshape(B * F), wtiled.reshape(F * V), bias16)
    return out.reshape(B, 1)
